# Initial kernel scaffold; baseline (speedup 1.0000x reference)
import jax
import jax.numpy as jnp
from jax.experimental import pallas as pl


def kernel(v, a, epsilon, W1, b1, g1, be1, W2, b2, g2, be2):
    raise NotImplementedError("write your pallas kernel here")


# speedup vs baseline: 1.2613x; 1.2613x over previous
import jax
import jax.numpy as jnp
from jax.experimental import pallas as pl

_BM = 256


def _ln_relu(x, g, b, eps=1e-5):
    mu = jnp.mean(x, axis=-1, keepdims=True)
    var = jnp.mean((x - mu) ** 2, axis=-1, keepdims=True)
    y = (x - mu) * jax.lax.rsqrt(var + eps) * g + b
    return jnp.maximum(y, 0.0)


def _mm_kernel(v_ref, w1t_ref, u_ref):
    u_ref[...] = jnp.dot(v_ref[...], w1t_ref[...],
                         preferred_element_type=jnp.float32)


def _gin_kernel(a_ref, u_ref, ublk_ref, eps_ref, b1_ref, g1_ref, be1_ref,
                w2t_ref, b2_ref, g2_ref, be2_ref, o_ref):
    h = jnp.dot(a_ref[...], u_ref[...], preferred_element_type=jnp.float32)
    h = h + eps_ref[0, 0] * ublk_ref[...] + b1_ref[...]
    h = _ln_relu(h, g1_ref[...], be1_ref[...])
    h2 = jnp.dot(h, w2t_ref[...], preferred_element_type=jnp.float32)
    h2 = h2 + b2_ref[...]
    o_ref[...] = _ln_relu(h2, g2_ref[...], be2_ref[...])


def kernel(v, a, epsilon, W1, b1, g1, be1, W2, b2, g2, be2):
    n, _ = a.shape
    hid = W1.shape[0]
    out_dim = W2.shape[0]
    grid = (n // _BM,)

    w1t = W1.T
    u = pl.pallas_call(
        _mm_kernel,
        grid=grid,
        in_specs=[
            pl.BlockSpec((_BM, n), lambda i: (i, 0)),
            pl.BlockSpec((n, hid), lambda i: (0, 0)),
        ],
        out_specs=pl.BlockSpec((_BM, hid), lambda i: (i, 0)),
        out_shape=jax.ShapeDtypeStruct((n, hid), jnp.float32),
    )(v, w1t)

    row = lambda x: x.reshape(1, -1)
    out = pl.pallas_call(
        _gin_kernel,
        grid=grid,
        in_specs=[
            pl.BlockSpec((_BM, n), lambda i: (i, 0)),
            pl.BlockSpec((n, hid), lambda i: (0, 0)),
            pl.BlockSpec((_BM, hid), lambda i: (i, 0)),
            pl.BlockSpec((1, 1), lambda i: (0, 0)),
            pl.BlockSpec((1, hid), lambda i: (0, 0)),
            pl.BlockSpec((1, hid), lambda i: (0, 0)),
            pl.BlockSpec((1, hid), lambda i: (0, 0)),
            pl.BlockSpec((hid, out_dim), lambda i: (0, 0)),
            pl.BlockSpec((1, out_dim), lambda i: (0, 0)),
            pl.BlockSpec((1, out_dim), lambda i: (0, 0)),
            pl.BlockSpec((1, out_dim), lambda i: (0, 0)),
        ],
        out_specs=pl.BlockSpec((_BM, out_dim), lambda i: (i, 0)),
        out_shape=jax.ShapeDtypeStruct((n, out_dim), jnp.float32),
    )(a, u, u, epsilon, row(b1), row(g1), row(be1),
      W2.T, row(b2), row(g2), row(be2))
    return out


# speedup vs baseline: 1.3875x; 1.1001x over previous
import functools

import jax
import jax.numpy as jnp
from jax.experimental import pallas as pl
from jax.experimental.pallas import tpu as pltpu

_BM = 256


def _ln_relu(x, g, b, eps=1e-5):
    mu = jnp.mean(x, axis=-1, keepdims=True)
    var = jnp.mean((x - mu) ** 2, axis=-1, keepdims=True)
    y = (x - mu) * jax.lax.rsqrt(var + eps) * g + b
    return jnp.maximum(y, 0.0)


def _fused_kernel(v_ref, a_ref, eps_ref, w1t_ref, b1_ref, g1_ref, be1_ref,
                  w2t_ref, b2_ref, g2_ref, be2_ref, o_ref, u_ref, *, nb):
    i = pl.program_id(0)

    @pl.when(i < nb)
    def _phase_mm():
        u_ref[pl.ds(i * _BM, _BM), :] = jnp.dot(
            v_ref[...], w1t_ref[...], preferred_element_type=jnp.float32)

    @pl.when(i >= nb)
    def _phase_gin():
        j = i - nb
        h = jnp.dot(a_ref[...], u_ref[...], preferred_element_type=jnp.float32)
        h = h + eps_ref[0, 0] * u_ref[pl.ds(j * _BM, _BM), :] + b1_ref[...]
        h = _ln_relu(h, g1_ref[...], be1_ref[...])
        h2 = jnp.dot(h, w2t_ref[...], preferred_element_type=jnp.float32)
        h2 = h2 + b2_ref[...]
        o_ref[...] = _ln_relu(h2, g2_ref[...], be2_ref[...])


def kernel(v, a, epsilon, W1, b1, g1, be1, W2, b2, g2, be2):
    n, _ = a.shape
    hid = W1.shape[0]
    out_dim = W2.shape[0]
    nb = n // _BM

    row = lambda x: x.reshape(1, -1)
    const = lambda i: (0, 0)
    out = pl.pallas_call(
        functools.partial(_fused_kernel, nb=nb),
        grid=(2 * nb,),
        in_specs=[
            pl.BlockSpec((_BM, n), lambda i: (jnp.minimum(i, nb - 1), 0)),
            pl.BlockSpec((_BM, n), lambda i: (jnp.maximum(i - nb, 0), 0)),
            pl.BlockSpec((1, 1), const),
            pl.BlockSpec((n, hid), const),
            pl.BlockSpec((1, hid), const),
            pl.BlockSpec((1, hid), const),
            pl.BlockSpec((1, hid), const),
            pl.BlockSpec((hid, out_dim), const),
            pl.BlockSpec((1, out_dim), const),
            pl.BlockSpec((1, out_dim), const),
            pl.BlockSpec((1, out_dim), const),
        ],
        out_specs=pl.BlockSpec((_BM, out_dim), lambda i: (jnp.maximum(i - nb, 0), 0)),
        out_shape=jax.ShapeDtypeStruct((n, out_dim), jnp.float32),
        scratch_shapes=[pltpu.VMEM((n, hid), jnp.float32)],
    )(v, a, epsilon, W1.T, row(b1), row(g1), row(be1),
      W2.T, row(b2), row(g2), row(be2))
    return out
